# baseline (speedup 1.0000x reference)
import functools

import jax
import jax.numpy as jnp
from jax import lax
from jax.experimental import pallas as pl
from jax.experimental.pallas import tpu as pltpu
from jax.experimental.pallas import tpu_sc as plsc

N_ROWS = 32768
N_COLS = 64
K = 8
NC = 2
NS = 16
L = 16
NW = NC * NS
RPW = N_ROWS // NW
GROUPS = RPW // L


def _sc_body(logits_hbm, idx_hbm, prob_hbm, vals_v, idx_v, val_v, prob_v):
    wid = lax.axis_index("s") * NC + lax.axis_index("c")
    pltpu.sync_copy(logits_hbm.at[pl.ds(wid * (RPW * N_COLS), RPW * N_COLS)],
                    vals_v)

    lane = lax.iota(jnp.int32, L)
    neg = jnp.full((L,), -jnp.inf, jnp.float32)
    cint = [jnp.full((L,), j, jnp.int32) for j in range(N_COLS)]

    def one_group(g):
        vbase = (g * L + lane) * N_COLS
        obase = (g * L + lane) * K

        regs = [neg] * K
        for j in range(N_COLS):
            x = plsc.load_gather(vals_v, [vbase + cint[j]])
            gt = [x > r for r in regs]
            new_regs = []
            for i in range(K):
                shifted = jnp.where(gt[i + 1], regs[i + 1], x) if i + 1 < K else x
                new_regs.append(jnp.where(gt[i], shifted, regs[i]))
            regs = new_regs
        t = regs[0]
        m = regs[K - 1]

        eq_budget = jnp.zeros((L,), jnp.int32)
        for r in regs:
            eq_budget = eq_budget + jnp.where(r == t, 1, 0)
        eq_seen = jnp.zeros((L,), jnp.int32)
        cnt = obase
        cap = obase + (K - 1)
        for j in range(N_COLS):
            x = plsc.load_gather(vals_v, [vbase + cint[j]])
            is_eq = x == t
            sel = jnp.logical_or(x > t,
                                 jnp.logical_and(is_eq, eq_seen < eq_budget))
            pos = jnp.minimum(cnt, cap)
            plsc.store_scatter(idx_v, [pos], cint[j], mask=sel)
            plsc.store_scatter(val_v, [pos], x, mask=sel)
            cnt = cnt + jnp.where(sel, 1, 0)
            eq_seen = eq_seen + jnp.where(is_eq, 1, 0)

        es = []
        denom = jnp.zeros((L,), jnp.float32)
        for p in range(K):
            vp = plsc.load_gather(val_v, [obase + cint[p]])
            e = jnp.exp(vp - m)
            es.append(e)
            denom = denom + e
        inv = 1.0 / denom
        for p in range(K):
            plsc.store_scatter(prob_v, [obase + cint[p]], es[p] * inv)

    G = 2

    def group(i, carry):
        for gg in range(G):
            one_group(i * G + gg)
        return carry

    lax.fori_loop(0, GROUPS // G, group, jnp.int32(0))

    pltpu.sync_copy(idx_v, idx_hbm.at[pl.ds(wid * (RPW * K), RPW * K)])
    pltpu.sync_copy(prob_v, prob_hbm.at[pl.ds(wid * (RPW * K), RPW * K)])


_sc_call = functools.partial(
    pl.kernel,
    out_type=(
        jax.ShapeDtypeStruct((N_ROWS * K,), jnp.int32),
        jax.ShapeDtypeStruct((N_ROWS * K,), jnp.float32),
    ),
    mesh=plsc.VectorSubcoreMesh(
        core_axis_name="c", subcore_axis_name="s",
        num_cores=NC, num_subcores=NS,
    ),
    compiler_params=pltpu.CompilerParams(needs_layout_passes=False),
    scratch_types=[
        pltpu.VMEM((RPW * N_COLS,), jnp.float32),
        pltpu.VMEM((RPW * K,), jnp.int32),
        pltpu.VMEM((RPW * K,), jnp.float32),
        pltpu.VMEM((RPW * K,), jnp.float32),
    ],
)(_sc_body)


def kernel(logits):
    flat_idx, flat_prob = _sc_call(logits.reshape(-1))
    return flat_idx.reshape(N_ROWS, K), flat_prob.reshape(N_ROWS, K)


# speedup vs baseline: 1.3875x; 1.3875x over previous
import functools

import jax
import jax.numpy as jnp
from jax import lax
from jax.experimental import pallas as pl
from jax.experimental.pallas import tpu as pltpu
from jax.experimental.pallas import tpu_sc as plsc

N_ROWS = 32768
N_COLS = 64
K = 8
VPAD = 65
OPAD = 9
NC = 2
NS = 16
L = 16
NW = NC * NS
RPW = N_ROWS // NW
GROUPS = RPW // L


def _sc_body(logits_hbm, idx_hbm, prob_hbm, vals_v, idx_v, val_v, prob_v):
    wid = lax.axis_index("s") * NC + lax.axis_index("c")
    base = wid * RPW
    pltpu.sync_copy(logits_hbm.at[pl.ds(base * VPAD, RPW * VPAD)], vals_v)

    lane = lax.iota(jnp.int32, L)
    neg = jnp.full((L,), -jnp.inf, jnp.float32)
    cint = [jnp.full((L,), j, jnp.int32) for j in range(N_COLS)]

    def group(g, carry):
        vbase = (g * L + lane) * VPAD
        obase = (g * L + lane) * OPAD

        regs = [neg] * K
        for j in range(N_COLS):
            x = plsc.load_gather(vals_v, [vbase + cint[j]])
            gt = [x > r for r in regs]
            new_regs = []
            for i in range(K):
                shifted = jnp.where(gt[i + 1], regs[i + 1], x) if i + 1 < K else x
                new_regs.append(jnp.where(gt[i], shifted, regs[i]))
            regs = new_regs
        t = regs[0]
        m = regs[K - 1]

        eq_budget = jnp.zeros((L,), jnp.int32)
        for r in regs:
            eq_budget = eq_budget + jnp.where(r == t, 1, 0)
        eq_seen = jnp.zeros((L,), jnp.int32)
        cnt = obase
        cap = obase + (K - 1)
        for j in range(N_COLS):
            x = plsc.load_gather(vals_v, [vbase + cint[j]])
            is_eq = x == t
            sel = jnp.logical_or(x > t,
                                 jnp.logical_and(is_eq, eq_seen < eq_budget))
            pos = jnp.minimum(cnt, cap)
            plsc.store_scatter(idx_v, [pos], cint[j], mask=sel)
            plsc.store_scatter(val_v, [pos], x, mask=sel)
            cnt = cnt + jnp.where(sel, 1, 0)
            eq_seen = eq_seen + jnp.where(is_eq, 1, 0)

        es = []
        denom = jnp.zeros((L,), jnp.float32)
        for p in range(K):
            vp = plsc.load_gather(val_v, [obase + cint[p]])
            e = jnp.exp(vp - m)
            es.append(e)
            denom = denom + e
        inv = 1.0 / denom
        for p in range(K):
            plsc.store_scatter(prob_v, [obase + cint[p]], es[p] * inv)
        return carry

    lax.fori_loop(0, GROUPS, group, jnp.int32(0))

    pltpu.sync_copy(idx_v, idx_hbm.at[pl.ds(base * OPAD, RPW * OPAD)])
    pltpu.sync_copy(prob_v, prob_hbm.at[pl.ds(base * OPAD, RPW * OPAD)])


_sc_call = functools.partial(
    pl.kernel,
    out_type=(
        jax.ShapeDtypeStruct((N_ROWS * OPAD,), jnp.int32),
        jax.ShapeDtypeStruct((N_ROWS * OPAD,), jnp.float32),
    ),
    mesh=plsc.VectorSubcoreMesh(
        core_axis_name="c", subcore_axis_name="s",
        num_cores=NC, num_subcores=NS,
    ),
    compiler_params=pltpu.CompilerParams(needs_layout_passes=False),
    scratch_types=[
        pltpu.VMEM((RPW * VPAD,), jnp.float32),
        pltpu.VMEM((RPW * OPAD,), jnp.int32),
        pltpu.VMEM((RPW * OPAD,), jnp.float32),
        pltpu.VMEM((RPW * OPAD,), jnp.float32),
    ],
)(_sc_body)


def kernel(logits):
    padded = jnp.pad(logits, ((0, 0), (0, VPAD - N_COLS))).reshape(-1)
    idx_p, prob_p = _sc_call(padded)
    return (idx_p.reshape(N_ROWS, OPAD)[:, :K],
            prob_p.reshape(N_ROWS, OPAD)[:, :K])


# speedup vs baseline: 1.4603x; 1.0525x over previous
import functools

import jax
import jax.numpy as jnp
from jax import lax
from jax.experimental import pallas as pl
from jax.experimental.pallas import tpu as pltpu
from jax.experimental.pallas import tpu_sc as plsc

N_ROWS = 32768
N_COLS = 64
K = 8
VPAD = 65
OPAD = 9
NC = 2
NS = 16
L = 16
NW = NC * NS
RPW = N_ROWS // NW
GROUPS = RPW // L


def _sc_body(logits_hbm, idx_hbm, prob_hbm, vals_v, idx_v, val_v, prob_v):
    wid = lax.axis_index("s") * NC + lax.axis_index("c")
    base = wid * RPW
    pltpu.sync_copy(logits_hbm.at[pl.ds(base * VPAD, RPW * VPAD)], vals_v)

    lane = lax.iota(jnp.int32, L)
    cint = [jnp.full((L,), j, jnp.int32) for j in range(N_COLS)]

    sort8_net = [(0, 1), (2, 3), (4, 5), (6, 7),
                 (0, 2), (1, 3), (4, 6), (5, 7),
                 (1, 2), (5, 6),
                 (0, 4), (1, 5), (2, 6), (3, 7),
                 (2, 4), (3, 5),
                 (1, 2), (3, 4), (5, 6)]
    bitonic8_net = [(0, 4), (1, 5), (2, 6), (3, 7),
                    (0, 2), (1, 3), (4, 6), (5, 7),
                    (0, 1), (2, 3), (4, 5), (6, 7)]

    def apply_net(v, net):
        for i, j in net:
            lo = jnp.minimum(v[i], v[j])
            hi = jnp.maximum(v[i], v[j])
            v[i], v[j] = lo, hi
        return v

    def group(g, carry):
        vbase = (g * L + lane) * VPAD
        obase = (g * L + lane) * OPAD

        def load_block(b):
            return [plsc.load_gather(vals_v, [vbase + cint[8 * b + u]])
                    for u in range(K)]

        run = apply_net(load_block(0), sort8_net)
        for b in range(1, N_COLS // K):
            blk = apply_net(load_block(b), sort8_net)
            c = [jnp.maximum(run[i], blk[K - 1 - i]) for i in range(K)]
            run = apply_net(c, bitonic8_net)
        t = run[0]
        m = run[K - 1]
        regs = run

        eq_budget = jnp.zeros((L,), jnp.int32)
        for r in regs:
            eq_budget = eq_budget + jnp.where(r == t, 1, 0)
        eq_seen = jnp.zeros((L,), jnp.int32)
        cnt = obase
        for j in range(N_COLS):
            x = plsc.load_gather(vals_v, [vbase + cint[j]])
            is_eq = x == t
            sel = jnp.logical_or(x > t,
                                 jnp.logical_and(is_eq, eq_seen < eq_budget))
            plsc.store_scatter(idx_v, [cnt], cint[j], mask=sel)
            plsc.store_scatter(val_v, [cnt], x, mask=sel)
            cnt = cnt + jnp.where(sel, 1, 0)
            eq_seen = eq_seen + jnp.where(is_eq, 1, 0)

        es = []
        denom = jnp.zeros((L,), jnp.float32)
        for p in range(K):
            vp = plsc.load_gather(val_v, [obase + cint[p]])
            e = jnp.exp(vp - m)
            es.append(e)
            denom = denom + e
        inv = 1.0 / denom
        for p in range(K):
            plsc.store_scatter(prob_v, [obase + cint[p]], es[p] * inv)
        return carry

    lax.fori_loop(0, GROUPS, group, jnp.int32(0))

    pltpu.sync_copy(idx_v, idx_hbm.at[pl.ds(base * OPAD, RPW * OPAD)])
    pltpu.sync_copy(prob_v, prob_hbm.at[pl.ds(base * OPAD, RPW * OPAD)])


_sc_call = functools.partial(
    pl.kernel,
    out_type=(
        jax.ShapeDtypeStruct((N_ROWS * OPAD,), jnp.int32),
        jax.ShapeDtypeStruct((N_ROWS * OPAD,), jnp.float32),
    ),
    mesh=plsc.VectorSubcoreMesh(
        core_axis_name="c", subcore_axis_name="s",
        num_cores=NC, num_subcores=NS,
    ),
    compiler_params=pltpu.CompilerParams(needs_layout_passes=False),
    scratch_types=[
        pltpu.VMEM((RPW * VPAD,), jnp.float32),
        pltpu.VMEM((RPW * OPAD,), jnp.int32),
        pltpu.VMEM((RPW * OPAD,), jnp.float32),
        pltpu.VMEM((RPW * OPAD,), jnp.float32),
    ],
)(_sc_body)


def kernel(logits):
    padded = jnp.pad(logits, ((0, 0), (0, VPAD - N_COLS))).reshape(-1)
    idx_p, prob_p = _sc_call(padded)
    return (idx_p.reshape(N_ROWS, OPAD)[:, :K],
            prob_p.reshape(N_ROWS, OPAD)[:, :K])
